# baseline (speedup 1.0000x reference)
import functools

import jax
import jax.numpy as jnp
from jax import lax
from jax.experimental import pallas as pl
from jax.experimental.pallas import tpu as pltpu
from jax.experimental.pallas import tpu_sc as plsc

DIM = 64
SEN = 200
NC, NS = 2, 16
NW = NC * NS
GSZ = 100
GPB = SEN // GSZ


def _sc_embed(idx, W, pos, blks_per_w):
    @functools.partial(
        pl.kernel,
        out_type=jax.ShapeDtypeStruct((NW, blks_per_w, SEN, DIM), jnp.float32),
        mesh=plsc.VectorSubcoreMesh(core_axis_name="c", subcore_axis_name="s"),
        scratch_types=[
            pltpu.VMEM((blks_per_w * GPB, GSZ), jnp.int32),
            pltpu.VMEM((SEN, DIM), jnp.float32),
            pltpu.VMEM((SEN, DIM), jnp.float32),
            pltpu.SemaphoreType.DMA,
        ],
        compiler_params=pltpu.CompilerParams(use_tc_tiling_on_sc=False),
    )
    def k(idx_hbm, w_hbm, pos_hbm, out_hbm, idx_v, pos_v, rows_v, sem):
        wid = lax.axis_index("s") * NC + lax.axis_index("c")
        pltpu.async_copy(idx_hbm.at[wid], idx_v, sem).wait()
        pltpu.async_copy(pos_hbm, pos_v, sem).wait()

        @pl.loop(0, blks_per_w)
        def _(b):
            cps = [
                pltpu.async_copy(
                    w_hbm.at[idx_v.at[b * GPB + h]],
                    rows_v.at[pl.ds(h * GSZ, GSZ)],
                    sem,
                )
                for h in range(GPB)
            ]
            for c in cps:
                c.wait()

            @pl.loop(0, SEN)
            def _(i):
                for j in range(DIM // 16):
                    sl = pl.ds(j * 16, 16)
                    rows_v[i, sl] = rows_v[i, sl] + pos_v[i, sl]

            pltpu.async_copy(rows_v, out_hbm.at[wid, b], sem).wait()

    return k(idx, W, pos)


def kernel(inputs, W, pos_table):
    B, S = inputs.shape
    blks_per_w = (B * S) // (NW * SEN)
    idx = inputs.reshape(NW, blks_per_w * GPB, GSZ)
    pos = pos_table[1 : S + 1]
    out = _sc_embed(idx, W, pos, blks_per_w)
    return out.reshape(B, S, DIM)


# speedup vs baseline: 1.1531x; 1.1531x over previous
import functools

import jax
import jax.numpy as jnp
from jax import lax
from jax.experimental import pallas as pl
from jax.experimental.pallas import tpu as pltpu
from jax.experimental.pallas import tpu_sc as plsc

DIM = 64
SEN = 200
NC, NS = 2, 16
NW = NC * NS
GSZ = 100
GPB = SEN // GSZ


def _sc_embed(idx, W, pos, blks_per_w):
    NBUF = 4
    LOOK = 2

    @functools.partial(
        pl.kernel,
        out_type=jax.ShapeDtypeStruct((NW, blks_per_w, SEN, DIM), jnp.float32),
        mesh=plsc.VectorSubcoreMesh(core_axis_name="c", subcore_axis_name="s"),
        scratch_types=[
            pltpu.VMEM((blks_per_w * GPB, GSZ), jnp.int32),
            pltpu.VMEM((SEN, DIM), jnp.float32),
            pltpu.VMEM((NBUF, SEN, DIM), jnp.float32),
        ]
        + [pltpu.SemaphoreType.DMA] * (2 * NBUF),
        compiler_params=pltpu.CompilerParams(use_tc_tiling_on_sc=False),
    )
    def k(idx_hbm, w_hbm, pos_hbm, out_hbm, idx_v, pos_v, rows_v, *sems):
        gsem, wsem = sems[:NBUF], sems[NBUF:]
        wid = lax.axis_index("s") * NC + lax.axis_index("c")
        pltpu.async_copy(idx_hbm.at[wid], idx_v, gsem[0]).wait()
        pltpu.async_copy(pos_hbm, pos_v, gsem[0]).wait()

        def start_gather(blk, slot):
            for h in range(GPB):
                pltpu.async_copy(
                    w_hbm.at[idx_v.at[blk * GPB + h]],
                    rows_v.at[slot, pl.ds(h * GSZ, GSZ)],
                    gsem[slot],
                )

        def wait_gather(slot):
            pltpu.make_async_copy(
                w_hbm.at[pl.ds(0, SEN)], rows_v.at[slot], gsem[slot]
            ).wait()

        def wait_write(slot):
            pltpu.make_async_copy(
                rows_v.at[slot], out_hbm.at[wid, 0], wsem[slot]
            ).wait()

        for b in range(LOOK):
            start_gather(b, b)

        @pl.loop(0, blks_per_w, step=NBUF)
        def _(b0):
            for s in range(NBUF):
                blk = b0 + s
                wait_gather(s)

                @pl.loop(0, SEN)
                def _(i):
                    for j in range(DIM // 16):
                        sl = pl.ds(j * 16, 16)
                        rows_v[s, i, sl] = rows_v[s, i, sl] + pos_v[i, sl]

                pltpu.async_copy(rows_v.at[s], out_hbm.at[wid, blk], wsem[s])

                gblk = blk + LOOK
                gslot = (s + LOOK) % NBUF

                @pl.when(gblk < blks_per_w)
                def _():
                    @pl.when(gblk >= NBUF)
                    def _():
                        wait_write(gslot)

                    start_gather(gblk, gslot)

        for s in range(NBUF):
            wait_write(s)

    return k(idx, W, pos)


def kernel(inputs, W, pos_table):
    B, S = inputs.shape
    blks_per_w = (B * S) // (NW * SEN)
    idx = inputs.reshape(NW, blks_per_w * GPB, GSZ)
    pos = pos_table[1 : S + 1]
    out = _sc_embed(idx, W, pos, blks_per_w)
    return out.reshape(B, S, DIM)
